# baseline (speedup 1.0000x reference)
import functools

import numpy as np

import jax
import jax.numpy as jnp
from jax import lax
from jax.experimental import pallas as pl
from jax.experimental.pallas import tpu as pltpu
from jax.experimental.pallas import tpu_sc as plsc

N = 10000
E = 320000
D = 128
DE = 16
G = 16
NGF = 8
DN = 256
OUT = 4
EPS = 1e-7

NSC = 2
NT = 16
CH = 64
TILE_EDGES = 10240
EP = NSC * NT * TILE_EDGES
PAD = EP - E
ACC_ROWS = 10240
ROWS_PER_TILE = ACC_ROWS // NT
HALF = 64
CHUNKS = EP // CH // NT
BULK = 80
NPAIR = BULK // 2
NBULK = CHUNKS // BULK

_PERM = np.empty(D, np.int32)
for _g in range(0, D, 32):
    for _i in range(16):
        _PERM[_g + 2 * _i] = _g + _i
        _PERM[_g + 2 * _i + 1] = _g + 16 + _i

NBLK = 10
BN = N // NBLK
BE = 2560
EBLK = EP // BE


def _conv_sc_body(h2f, h2, ea_hbm, srcp2, dstp, out,
                  acc, src_all, dst_all, g_v, ea_v, upd_v, fin_v,
                  sg0, sg1, se0, se1, ss0, ss1):
    cid = lax.axis_index("c")
    sid = lax.axis_index("s")

    sgs = (sg0, sg1)
    ses = (se0, se1)
    sss = (ss0, ss1)
    z16 = jnp.zeros((16,), jnp.float32)

    def zb(i, c):
        for j in range(8):
            upd_v[0, i, pl.ds(j * 16, 16)] = z16
        return c
    lax.fori_loop(0, CH, zb, 0)

    def zc(r, c):
        pltpu.sync_copy(upd_v.at[0], acc.at[pl.ds(sid * ROWS_PER_TILE + r * CH, CH)])
        return c
    lax.fori_loop(0, ROWS_PER_TILE // CH, zc, 0)
    plsc.subcore_barrier()

    for bulk in range(NBULK):
        row0 = sid * CHUNKS + bulk * BULK
        pltpu.sync_copy(srcp2.at[cid, pl.ds(row0, BULK)], src_all)
        pltpu.sync_copy(dstp.at[pl.ds(row0, BULK)], dst_all)

        def fetch(c, slot):
            pltpu.async_copy(h2.at[src_all.at[c]], g_v.at[slot], sgs[slot])
            pltpu.async_copy(
                ea_hbm.at[pl.ds((row0 + c) * CH, CH), pl.ds(cid * (HALF // 2), HALF // 2)],
                ea_v.at[slot], ses[slot])

        def wait_fetch(slot):
            pltpu.make_async_copy(h2.at[src_all.at[0]], g_v.at[slot], sgs[slot]).wait()
            pltpu.make_async_copy(
                ea_hbm.at[pl.ds(0, CH), pl.ds(cid * (HALF // 2), HALF // 2)],
                ea_v.at[slot], ses[slot]).wait()

        def compute(slot):
            @plsc.parallel_loop(0, CH, step=1, unroll=4)
            def _(e):
                for jj in range(2):
                    gu = g_v[slot, e, pl.ds(jj * 16, 16)]
                    au = ea_v[slot, e, pl.ds(jj * 16, 16)]
                    g0 = lax.bitcast_convert_type(jnp.left_shift(gu, 16), jnp.float32)
                    g1 = lax.bitcast_convert_type(jnp.bitwise_and(gu, jnp.int32(-65536)), jnp.float32)
                    a0 = lax.bitcast_convert_type(jnp.left_shift(au, 16), jnp.float32)
                    a1 = lax.bitcast_convert_type(jnp.bitwise_and(au, jnp.int32(-65536)), jnp.float32)
                    for k, (gg, aa) in enumerate(((g0, a0), (g1, a1))):
                        t = jnp.maximum(gg + aa, 0.0) + EPS
                        p = jnp.exp(t)
                        off = jj * 32 + k * 16
                        upd_v[slot, e, pl.ds(off, 16)] = p
                        upd_v[slot, e, pl.ds(HALF + off, 16)] = t * p

        def scatter(c, slot):
            pltpu.async_copy(upd_v.at[slot], acc.at[dst_all.at[c]], sss[slot], add=True)

        def wait_scatter(slot):
            pltpu.make_async_copy(upd_v.at[slot], acc.at[dst_all.at[0]], sss[slot]).wait()

        fetch(0, 0)

        def pair(it, c):
            c0 = it * 2
            fetch(c0 + 1, 1)
            wait_fetch(0)

            @pl.when(it > 0)
            def _():
                wait_scatter(0)
            compute(0)
            scatter(c0, 0)

            @pl.when(it < NPAIR - 1)
            def _():
                fetch(c0 + 2, 0)
            wait_fetch(1)

            @pl.when(it > 0)
            def _():
                wait_scatter(1)
            compute(1)
            scatter(c0 + 1, 1)
            return c
        lax.fori_loop(0, NPAIR, pair, 0)
        wait_scatter(0)
        wait_scatter(1)
    plsc.subcore_barrier()

    FR = 40

    def fin(r, c):
        row = sid * ROWS_PER_TILE + r * FR

        @pl.when(row < N)
        def _():
            pltpu.sync_copy(acc.at[pl.ds(row, FR)], upd_v.at[0, pl.ds(0, FR)])
            pltpu.sync_copy(h2f.at[pl.ds(cid * N + row, FR)], fin_v)

            @plsc.parallel_loop(0, FR, step=1, unroll=4)
            def _(e):
                for j in range(4):
                    s = upd_v[0, e, pl.ds(j * 16, 16)]
                    w = upd_v[0, e, pl.ds(HALF + j * 16, 16)]
                    hv = fin_v[e, pl.ds(j * 16, 16)]
                    fin_v[e, pl.ds(j * 16, 16)] = hv + w / jnp.maximum(s, 1e-30)
            pltpu.sync_copy(fin_v, out.at[cid, pl.ds(row, FR)])
        return c
    lax.fori_loop(0, ROWS_PER_TILE // FR, fin, 0)


_conv_sc = functools.partial(
    pl.kernel,
    out_type=jax.ShapeDtypeStruct((NSC, N, HALF), jnp.float32),
    mesh=plsc.VectorSubcoreMesh(core_axis_name="c", subcore_axis_name="s"),
    scratch_types=[
        pltpu.VMEM_SHARED((ACC_ROWS, D), jnp.float32),
        pltpu.VMEM((BULK, CH), jnp.int32),
        pltpu.VMEM((BULK, CH), jnp.int32),
        pltpu.VMEM((2, CH, HALF // 2), jnp.int32),
        pltpu.VMEM((2, CH, HALF // 2), jnp.int32),
        pltpu.VMEM((2, CH, D), jnp.float32),
        pltpu.VMEM((40, HALF), jnp.float32),
        pltpu.SemaphoreType.DMA,
        pltpu.SemaphoreType.DMA,
        pltpu.SemaphoreType.DMA,
        pltpu.SemaphoreType.DMA,
        pltpu.SemaphoreType.DMA,
        pltpu.SemaphoreType.DMA,
    ],
    compiler_params=pltpu.CompilerParams(use_tc_tiling_on_sc=False),
)(_conv_sc_body)


def _ea_body(eat_ref, W_ref, b_ref, o_ref):
    a = eat_ref[...].astype(jnp.bfloat16)
    w = W_ref[...].astype(jnp.bfloat16)
    r = lax.dot_general(a, w, (((0,), (1,)), ((), ())),
                        preferred_element_type=jnp.float32)
    o_ref[...] = (r + b_ref[...]).astype(jnp.bfloat16)


_ea_tc = pl.pallas_call(
    _ea_body,
    grid=(EBLK,),
    in_specs=[
        pl.BlockSpec((DE, BE), lambda i: (0, jnp.minimum(i, E // BE - 1))),
        pl.BlockSpec((D, DE), lambda i: (0, 0)),
        pl.BlockSpec((D,), lambda i: (0,)),
    ],
    out_specs=pl.BlockSpec((BE, D), lambda i: (i, 0)),
    out_shape=jax.ShapeDtypeStruct((EP, D), jnp.bfloat16),
)


def _h_body(x_ref, W_ref, b_ref, Wp_ref, bp_ref, o_ref, ob_ref):
    xv = x_ref[...]
    r = jnp.dot(xv, W_ref[...].T, preferred_element_type=jnp.float32) + b_ref[...]
    o_ref[0] = r[:, :HALF]
    o_ref[1] = r[:, HALF:]
    rb = (jnp.dot(xv, Wp_ref[...].T, preferred_element_type=jnp.float32)
          + bp_ref[...]).astype(jnp.bfloat16)
    ob_ref[0] = rb[:, :HALF]
    ob_ref[1] = rb[:, HALF:]


_h_tc = pl.pallas_call(
    _h_body,
    grid=(NBLK,),
    in_specs=[
        pl.BlockSpec((BN, D), lambda i: (i, 0)),
        pl.BlockSpec((D, D), lambda i: (0, 0)),
        pl.BlockSpec((D,), lambda i: (0,)),
        pl.BlockSpec((D, D), lambda i: (0, 0)),
        pl.BlockSpec((D,), lambda i: (0,)),
    ],
    out_specs=[
        pl.BlockSpec((NSC, BN, HALF), lambda i: (0, i, 0)),
        pl.BlockSpec((NSC, BN, HALF), lambda i: (0, i, 0)),
    ],
    out_shape=[
        jax.ShapeDtypeStruct((NSC, N, HALF), jnp.float32),
        jax.ShapeDtypeStruct((NSC, N, HALF), jnp.bfloat16),
    ],
)


def _combine_body(o2_ref, W1_ref, b1_ref, W2_ref, b2_ref, W2p_ref, b2p_ref,
                  o_ref, ob_ref):
    o = jnp.concatenate([o2_ref[0], o2_ref[1]], axis=1)
    h1 = jax.nn.relu(jnp.dot(o, W1_ref[...].T, preferred_element_type=jnp.float32) + b1_ref[...])
    h2 = jax.nn.relu(jnp.dot(h1, W2_ref[...].T, preferred_element_type=jnp.float32) + b2_ref[...])
    o_ref[0] = h2[:, :HALF]
    o_ref[1] = h2[:, HALF:]
    hb = jax.nn.relu(jnp.dot(h1, W2p_ref[...].T, preferred_element_type=jnp.float32)
                     + b2p_ref[...]).astype(jnp.bfloat16)
    ob_ref[0] = hb[:, :HALF]
    ob_ref[1] = hb[:, HALF:]


_combine_tc = pl.pallas_call(
    _combine_body,
    grid=(NBLK,),
    in_specs=[
        pl.BlockSpec((NSC, BN, HALF), lambda i: (0, i, 0)),
        pl.BlockSpec((2 * D, D), lambda i: (0, 0)),
        pl.BlockSpec((2 * D,), lambda i: (0,)),
        pl.BlockSpec((D, 2 * D), lambda i: (0, 0)),
        pl.BlockSpec((D,), lambda i: (0,)),
        pl.BlockSpec((D, 2 * D), lambda i: (0, 0)),
        pl.BlockSpec((D,), lambda i: (0,)),
    ],
    out_specs=[
        pl.BlockSpec((NSC, BN, HALF), lambda i: (0, i, 0)),
        pl.BlockSpec((NSC, BN, HALF), lambda i: (0, i, 0)),
    ],
    out_shape=[
        jax.ShapeDtypeStruct((NSC, N, HALF), jnp.float32),
        jax.ShapeDtypeStruct((NSC, N, HALF), jnp.bfloat16),
    ],
)


def _head_body(h2_ref, b_ref, ga_ref, d1W_ref, d1b_ref, d2W_ref,
               d2b_ref, oW_ref, ob_ref, out_ref, pooled, cnt):
    i = pl.program_id(0)

    @pl.when(i == 0)
    def _():
        pooled[...] = jnp.zeros_like(pooled)
        cnt[...] = jnp.zeros_like(cnt)

    b = b_ref[0, 0, :]
    oh = (b[None, :] == lax.broadcasted_iota(jnp.int32, (G, BN), 0).astype(jnp.float32)).astype(jnp.float32)
    hblk = jnp.concatenate([h2_ref[0], h2_ref[1]], axis=1)
    pooled[...] += jnp.dot(oh, hblk, preferred_element_type=jnp.float32)
    cnt[...] += jnp.dot(oh, jnp.ones_like(hblk), preferred_element_type=jnp.float32)

    @pl.when(i == pl.num_programs(0) - 1)
    def _():
        pm = pooled[...] / jnp.maximum(cnt[...], 1.0)
        g = jnp.concatenate([pm, ga_ref[...]], axis=1)
        g = jax.nn.relu(jnp.dot(g, d1W_ref[...].T, preferred_element_type=jnp.float32) + d1b_ref[...])
        g = jax.nn.relu(jnp.dot(g, d2W_ref[...].T, preferred_element_type=jnp.float32) + d2b_ref[...])
        out_ref[...] = jax.nn.sigmoid(jnp.dot(g, oW_ref[...].T, preferred_element_type=jnp.float32) + ob_ref[...])


_head_tc = pl.pallas_call(
    _head_body,
    grid=(NBLK,),
    in_specs=[
        pl.BlockSpec((NSC, BN, HALF), lambda i: (0, i, 0)),
        pl.BlockSpec((1, 1, BN), lambda i: (i, 0, 0)),
        pl.BlockSpec((G, NGF), lambda i: (0, 0)),
        pl.BlockSpec((DN, D + NGF), lambda i: (0, 0)),
        pl.BlockSpec((DN,), lambda i: (0,)),
        pl.BlockSpec((DN, DN), lambda i: (0, 0)),
        pl.BlockSpec((DN,), lambda i: (0,)),
        pl.BlockSpec((OUT, DN), lambda i: (0, 0)),
        pl.BlockSpec((OUT,), lambda i: (0,)),
    ],
    out_specs=pl.BlockSpec((G, OUT), lambda i: (0, 0)),
    out_shape=jax.ShapeDtypeStruct((G, OUT), jnp.float32),
    scratch_shapes=[
        pltpu.VMEM((G, D), jnp.float32),
        pltpu.VMEM((G, D), jnp.float32),
    ],
)


def kernel(x, edge_index, edge_attr, graph_attr, batch, node_W, node_b,
           edge_W, edge_b, c1_W1, c1_b1, c1_W2, c1_b2, c2_W1, c2_b1, c2_W2,
           c2_b2, c3_W1, c3_b1, c3_W2, c3_b2, d1_W, d1_b, d2_W, d2_b, o_W, o_b):
    src = edge_index[0]
    dst = edge_index[1]
    ar = jnp.arange(PAD, dtype=jnp.int32)
    srcp = jnp.concatenate([src, (ar * 37) % N])
    srcp2 = jnp.stack([srcp, srcp + N]).reshape(NSC, EP // CH, CH)
    dstp = jnp.concatenate([dst, N + (ar % (ACC_ROWS - N))]).reshape(EP // CH, CH)
    batch_r = batch.astype(jnp.float32).reshape(NBLK, 1, BN)

    ea = _ea_tc(edge_attr.T, edge_W[_PERM], edge_b[_PERM])
    ea_i = lax.bitcast_convert_type(ea.reshape(EP, D // 2, 2), jnp.int32)
    h2, h2b = _h_tc(x, node_W, node_b, node_W[_PERM], node_b[_PERM])
    for W1, b1, W2, b2 in ((c1_W1, c1_b1, c1_W2, c1_b2),
                           (c2_W1, c2_b1, c2_W2, c2_b2),
                           (c3_W1, c3_b1, c3_W2, c3_b2)):
        h2b_i = lax.bitcast_convert_type(
            h2b.reshape(NSC * N, HALF // 2, 2), jnp.int32)
        part = _conv_sc(h2.reshape(NSC * N, HALF), h2b_i, ea_i, srcp2, dstp)
        h2, h2b = _combine_tc(part, W1, b1, W2, b2, W2[_PERM], b2[_PERM])
    return _head_tc(h2, batch_r, graph_attr, d1_W, d1_b, d2_W, d2_b,
                    o_W, o_b)


# speedup vs baseline: 1.9199x; 1.9199x over previous
import functools

import jax
import jax.numpy as jnp
from jax import lax
from jax.experimental import pallas as pl
from jax.experimental.pallas import tpu as pltpu
from jax.experimental.pallas import tpu_sc as plsc

N = 10000
E = 320000
D = 128
DE = 16
G = 16
NGF = 8
DN = 256
OUT = 4
EPS = 1e-7

NSC = 2
NT = 16
CH = 64
TILE_EDGES = 10240
EP = NSC * NT * TILE_EDGES
PAD = EP - E
ACC_ROWS = 10240
ROWS_PER_TILE = ACC_ROWS // NT
HALF = 64
CHUNKS = EP // CH // NT
BULK = 80
NPAIR = BULK // 2
NBULK = CHUNKS // BULK

NBLK = 10
BN = N // NBLK
BE = 2560
EBLK = EP // BE


def _conv_sc_body(h2, ea_hbm, srcp2, dstp, out,
                  acc, src_all, dst_all, g_v, ea_v, upd_v, fin_v,
                  sg0, sg1, se0, se1, ss0, ss1):
    cid = lax.axis_index("c")
    sid = lax.axis_index("s")

    sgs = (sg0, sg1)
    ses = (se0, se1)
    sss = (ss0, ss1)
    z16 = jnp.zeros((16,), jnp.float32)

    def zb(i, c):
        for j in range(8):
            upd_v[0, i, pl.ds(j * 16, 16)] = z16
        return c
    lax.fori_loop(0, CH, zb, 0)

    def zc(r, c):
        pltpu.sync_copy(upd_v.at[0], acc.at[pl.ds(sid * ROWS_PER_TILE + r * CH, CH)])
        return c
    lax.fori_loop(0, ROWS_PER_TILE // CH, zc, 0)
    plsc.subcore_barrier()

    for bulk in range(NBULK):
        row0 = sid * CHUNKS + bulk * BULK
        pltpu.sync_copy(srcp2.at[cid, pl.ds(row0, BULK)], src_all)
        pltpu.sync_copy(dstp.at[pl.ds(row0, BULK)], dst_all)

        def fetch(c, slot):
            pltpu.async_copy(h2.at[src_all.at[c]], g_v.at[slot], sgs[slot])
            pltpu.async_copy(
                ea_hbm.at[pl.ds((row0 + c) * CH, CH), pl.ds(cid * HALF, HALF)],
                ea_v.at[slot], ses[slot])

        def wait_fetch(slot):
            pltpu.make_async_copy(h2.at[src_all.at[0]], g_v.at[slot], sgs[slot]).wait()
            pltpu.make_async_copy(
                ea_hbm.at[pl.ds(0, CH), pl.ds(cid * HALF, HALF)],
                ea_v.at[slot], ses[slot]).wait()

        def compute(slot):
            @plsc.parallel_loop(0, CH, step=1, unroll=4)
            def _(e):
                for j in range(4):
                    gv = g_v[slot, e, pl.ds(j * 16, 16)]
                    av = ea_v[slot, e, pl.ds(j * 16, 16)]
                    t = jnp.maximum(gv + av, 0.0) + EPS
                    p = jnp.exp(t)
                    upd_v[slot, e, pl.ds(j * 16, 16)] = p
                    upd_v[slot, e, pl.ds(HALF + j * 16, 16)] = t * p

        def scatter(c, slot):
            pltpu.async_copy(upd_v.at[slot], acc.at[dst_all.at[c]], sss[slot], add=True)

        def wait_scatter(slot):
            pltpu.make_async_copy(upd_v.at[slot], acc.at[dst_all.at[0]], sss[slot]).wait()

        fetch(0, 0)

        def pair(it, c):
            c0 = it * 2
            fetch(c0 + 1, 1)
            wait_fetch(0)

            @pl.when(it > 0)
            def _():
                wait_scatter(0)
            compute(0)
            scatter(c0, 0)

            @pl.when(it < NPAIR - 1)
            def _():
                fetch(c0 + 2, 0)
            wait_fetch(1)

            @pl.when(it > 0)
            def _():
                wait_scatter(1)
            compute(1)
            scatter(c0 + 1, 1)
            return c
        lax.fori_loop(0, NPAIR, pair, 0)
        wait_scatter(0)
        wait_scatter(1)
    plsc.subcore_barrier()

    FR = 40

    def fin(r, c):
        row = sid * ROWS_PER_TILE + r * FR

        @pl.when(row < N)
        def _():
            pltpu.sync_copy(acc.at[pl.ds(row, FR)], upd_v.at[0, pl.ds(0, FR)])
            pltpu.sync_copy(h2.at[pl.ds(cid * N + row, FR)], fin_v)

            @plsc.parallel_loop(0, FR, step=1, unroll=4)
            def _(e):
                for j in range(4):
                    s = upd_v[0, e, pl.ds(j * 16, 16)]
                    w = upd_v[0, e, pl.ds(HALF + j * 16, 16)]
                    hv = fin_v[e, pl.ds(j * 16, 16)]
                    fin_v[e, pl.ds(j * 16, 16)] = hv + w / jnp.maximum(s, 1e-30)
            pltpu.sync_copy(fin_v, out.at[cid, pl.ds(row, FR)])
        return c
    lax.fori_loop(0, ROWS_PER_TILE // FR, fin, 0)


_conv_sc = functools.partial(
    pl.kernel,
    out_type=jax.ShapeDtypeStruct((NSC, N, HALF), jnp.float32),
    mesh=plsc.VectorSubcoreMesh(core_axis_name="c", subcore_axis_name="s"),
    scratch_types=[
        pltpu.VMEM_SHARED((ACC_ROWS, D), jnp.float32),
        pltpu.VMEM((BULK, CH), jnp.int32),
        pltpu.VMEM((BULK, CH), jnp.int32),
        pltpu.VMEM((2, CH, HALF), jnp.float32),
        pltpu.VMEM((2, CH, HALF), jnp.float32),
        pltpu.VMEM((2, CH, D), jnp.float32),
        pltpu.VMEM((40, HALF), jnp.float32),
        pltpu.SemaphoreType.DMA,
        pltpu.SemaphoreType.DMA,
        pltpu.SemaphoreType.DMA,
        pltpu.SemaphoreType.DMA,
        pltpu.SemaphoreType.DMA,
        pltpu.SemaphoreType.DMA,
    ],
    compiler_params=pltpu.CompilerParams(use_tc_tiling_on_sc=False),
)(_conv_sc_body)


def _ea_body(eat_ref, W_ref, b_ref, o_ref):
    a = eat_ref[...].astype(jnp.bfloat16)
    w = W_ref[...].astype(jnp.bfloat16)
    r = lax.dot_general(a, w, (((0,), (1,)), ((), ())),
                        preferred_element_type=jnp.float32)
    o_ref[...] = r + b_ref[...]


_ea_tc = pl.pallas_call(
    _ea_body,
    grid=(EBLK,),
    in_specs=[
        pl.BlockSpec((DE, BE), lambda i: (0, jnp.minimum(i, E // BE - 1))),
        pl.BlockSpec((D, DE), lambda i: (0, 0)),
        pl.BlockSpec((D,), lambda i: (0,)),
    ],
    out_specs=pl.BlockSpec((BE, D), lambda i: (i, 0)),
    out_shape=jax.ShapeDtypeStruct((EP, D), jnp.float32),
)


def _h_body(x_ref, W_ref, b_ref, o_ref):
    r = jnp.dot(x_ref[...], W_ref[...].T, preferred_element_type=jnp.float32) + b_ref[...]
    o_ref[0] = r[:, :HALF]
    o_ref[1] = r[:, HALF:]


_h_tc = pl.pallas_call(
    _h_body,
    grid=(NBLK,),
    in_specs=[
        pl.BlockSpec((BN, D), lambda i: (i, 0)),
        pl.BlockSpec((D, D), lambda i: (0, 0)),
        pl.BlockSpec((D,), lambda i: (0,)),
    ],
    out_specs=pl.BlockSpec((NSC, BN, HALF), lambda i: (0, i, 0)),
    out_shape=jax.ShapeDtypeStruct((NSC, N, HALF), jnp.float32),
)


def _combine_body(o2_ref, W1_ref, b1_ref, W2_ref, b2_ref, o_ref):
    o = jnp.concatenate([o2_ref[0], o2_ref[1]], axis=1)
    h1 = jax.nn.relu(jnp.dot(o, W1_ref[...].T, preferred_element_type=jnp.float32) + b1_ref[...])
    h2 = jax.nn.relu(jnp.dot(h1, W2_ref[...].T, preferred_element_type=jnp.float32) + b2_ref[...])
    o_ref[0] = h2[:, :HALF]
    o_ref[1] = h2[:, HALF:]


_combine_tc = pl.pallas_call(
    _combine_body,
    grid=(NBLK,),
    in_specs=[
        pl.BlockSpec((NSC, BN, HALF), lambda i: (0, i, 0)),
        pl.BlockSpec((2 * D, D), lambda i: (0, 0)),
        pl.BlockSpec((2 * D,), lambda i: (0,)),
        pl.BlockSpec((D, 2 * D), lambda i: (0, 0)),
        pl.BlockSpec((D,), lambda i: (0,)),
    ],
    out_specs=pl.BlockSpec((NSC, BN, HALF), lambda i: (0, i, 0)),
    out_shape=jax.ShapeDtypeStruct((NSC, N, HALF), jnp.float32),
)


def _head_body(h2_ref, b_ref, ga_ref, d1W_ref, d1b_ref, d2W_ref,
               d2b_ref, oW_ref, ob_ref, out_ref, pooled, cnt):
    i = pl.program_id(0)

    @pl.when(i == 0)
    def _():
        pooled[...] = jnp.zeros_like(pooled)
        cnt[...] = jnp.zeros_like(cnt)

    b = b_ref[0, 0, :]
    oh = (b[None, :] == lax.broadcasted_iota(jnp.int32, (G, BN), 0).astype(jnp.float32)).astype(jnp.float32)
    hblk = jnp.concatenate([h2_ref[0], h2_ref[1]], axis=1)
    pooled[...] += jnp.dot(oh, hblk, preferred_element_type=jnp.float32)
    cnt[...] += jnp.dot(oh, jnp.ones_like(hblk), preferred_element_type=jnp.float32)

    @pl.when(i == pl.num_programs(0) - 1)
    def _():
        pm = pooled[...] / jnp.maximum(cnt[...], 1.0)
        g = jnp.concatenate([pm, ga_ref[...]], axis=1)
        g = jax.nn.relu(jnp.dot(g, d1W_ref[...].T, preferred_element_type=jnp.float32) + d1b_ref[...])
        g = jax.nn.relu(jnp.dot(g, d2W_ref[...].T, preferred_element_type=jnp.float32) + d2b_ref[...])
        out_ref[...] = jax.nn.sigmoid(jnp.dot(g, oW_ref[...].T, preferred_element_type=jnp.float32) + ob_ref[...])


_head_tc = pl.pallas_call(
    _head_body,
    grid=(NBLK,),
    in_specs=[
        pl.BlockSpec((NSC, BN, HALF), lambda i: (0, i, 0)),
        pl.BlockSpec((1, 1, BN), lambda i: (i, 0, 0)),
        pl.BlockSpec((G, NGF), lambda i: (0, 0)),
        pl.BlockSpec((DN, D + NGF), lambda i: (0, 0)),
        pl.BlockSpec((DN,), lambda i: (0,)),
        pl.BlockSpec((DN, DN), lambda i: (0, 0)),
        pl.BlockSpec((DN,), lambda i: (0,)),
        pl.BlockSpec((OUT, DN), lambda i: (0, 0)),
        pl.BlockSpec((OUT,), lambda i: (0,)),
    ],
    out_specs=pl.BlockSpec((G, OUT), lambda i: (0, 0)),
    out_shape=jax.ShapeDtypeStruct((G, OUT), jnp.float32),
    scratch_shapes=[
        pltpu.VMEM((G, D), jnp.float32),
        pltpu.VMEM((G, D), jnp.float32),
    ],
)


def kernel(x, edge_index, edge_attr, graph_attr, batch, node_W, node_b,
           edge_W, edge_b, c1_W1, c1_b1, c1_W2, c1_b2, c2_W1, c2_b1, c2_W2,
           c2_b2, c3_W1, c3_b1, c3_W2, c3_b2, d1_W, d1_b, d2_W, d2_b, o_W, o_b):
    src = edge_index[0]
    dst = edge_index[1]
    ar = jnp.arange(PAD, dtype=jnp.int32)
    srcp = jnp.concatenate([src, (ar * 37) % N])
    srcp2 = jnp.stack([srcp, srcp + N]).reshape(NSC, EP // CH, CH)
    dstp = jnp.concatenate([dst, N + (ar % (ACC_ROWS - N))]).reshape(EP // CH, CH)
    batch_r = batch.astype(jnp.float32).reshape(NBLK, 1, BN)

    ea = _ea_tc(edge_attr.T, edge_W, edge_b)
    h2 = _h_tc(x, node_W, node_b)
    for W1, b1, W2, b2 in ((c1_W1, c1_b1, c1_W2, c1_b2),
                           (c2_W1, c2_b1, c2_W2, c2_b2),
                           (c3_W1, c3_b1, c3_W2, c3_b2)):
        part = _conv_sc(h2.reshape(NSC * N, HALF), ea, srcp2, dstp)
        h2 = _combine_tc(part, W1, b1, W2, b2)
    return _head_tc(h2, batch_r, graph_attr, d1_W, d1_b, d2_W, d2_b,
                    o_W, o_b)
